# baseline (speedup 1.0000x reference)
import math

import jax
import jax.numpy as jnp
from jax.experimental import pallas as pl
from jax.experimental.pallas import tpu as pltpu

N = 2048
BI = 8
BJ = 512
SCALE = 1.0 / math.sqrt(2.0)


def _lrelu(x):
    return jnp.where(x >= 0, x, 0.2 * x)


def _group_norm_rows(x, gamma, beta, groups, eps=1e-5):
    c = x.shape[0]
    per = c // groups
    outs = []
    for g in range(groups):
        sub = x[g * per:(g + 1) * per, :]
        m = jnp.mean(sub)
        v = jnp.mean((sub - m) ** 2)
        outs.append((sub - m) / jnp.sqrt(v + eps))
    y = jnp.concatenate(outs, axis=0)
    return y * gamma + beta


def _prologue_body(featT_ref, ptsT_ref, pts_ref, normT_ref,
                   w1_ref, b1_ref, w2_ref, b2_ref, g_ref, be_ref,
                   jd_ref, pts_out_ref):
    f = jnp.dot(w1_ref[...], featT_ref[...],
                preferred_element_type=jnp.float32) + b1_ref[...]
    f = _lrelu(f)
    f = jnp.dot(w2_ref[...], f, preferred_element_type=jnp.float32) + b2_ref[...]
    f = _lrelu(f)
    f = _group_norm_rows(f, g_ref[...], be_ref[...], groups=4)
    jd = jnp.concatenate([
        ptsT_ref[...] * SCALE,
        normT_ref[...],
        jnp.zeros((2, N), jnp.float32),
        f,
    ], axis=0)
    jd_ref[...] = jd
    pts_out_ref[...] = pts_ref[...] * SCALE


def _pair_body(w1k_ref, w2k_ref, pts_i_ref, nuv_i_ref, jd_ref, out_ref):
    xi = [pts_i_ref[:, c:c + 1] for c in range(3)]
    frame = [[nuv_i_ref[:, 3 * k + c:3 * k + c + 1] for c in range(3)]
             for k in range(3)]
    ni = frame[0]
    w1k = w1k_ref[...]
    w2k = w2k_ref[...]
    ones = jnp.ones((BI, BJ), jnp.float32)
    accs = [jnp.zeros((BI, BJ), jnp.float32) for _ in range(8)]
    for blk in range(N // BJ):
        sl = slice(blk * BJ, (blk + 1) * BJ)
        xj = [jd_ref[c:c + 1, sl] for c in range(3)]
        nj = [jd_ref[3 + c:4 + c, sl] for c in range(3)]
        dx = [xj[c] - xi[c] for c in range(3)]
        ndot = ni[0] * nj[0] + ni[1] * nj[1] + ni[2] * nj[2]
        sq = dx[0] * dx[0] + dx[1] * dx[1] + dx[2] * dx[2]
        t = 2.0 - ndot
        w = jnp.exp(-(sq * t * t))
        x_loc = [frame[k][0] * dx[0] + frame[k][1] * dx[1] + frame[k][2] * dx[2]
                 for k in range(3)]
        x_stack = jnp.concatenate(x_loc + [ones], axis=0)
        cuts = jnp.maximum(
            jnp.dot(w1k, x_stack, preferred_element_type=jnp.float32), 0.0)
        cc = jnp.concatenate([cuts, ones], axis=0)
        hs = jnp.maximum(
            jnp.dot(w2k, cc, preferred_element_type=jnp.float32), 0.0)
        for h in range(8):
            hh = hs[8 * h:8 * h + 8, :]
            accs[h] = accs[h] + (w * hh) * jd_ref[8 + h:9 + h, sl]
    out_ref[...] = jnp.concatenate(
        [jnp.sum(a, axis=1, keepdims=True) for a in accs], axis=1)


def _epilogue_body(pT_ref, w1_ref, b1_ref, w2_ref, b2_ref, g_ref, be_ref,
                   out_ref):
    f = jnp.dot(w1_ref[...], pT_ref[...],
                preferred_element_type=jnp.float32) + b1_ref[...]
    f = _lrelu(f)
    f = jnp.dot(w2_ref[...], f, preferred_element_type=jnp.float32) + b2_ref[...]
    f = _lrelu(f)
    out_ref[...] = _group_norm_rows(f, g_ref[...], be_ref[...], groups=4)


def kernel(points, nuv, features, W_in1, b_in1, W_in2, b_in2, g_in, be_in,
           Wc1, bc1, Wc2, bc2, W_out1, b_out1, W_out2, b_out2, g_out, be_out):
    featT = features[0].T
    ptsT = points[0].T
    pts = points[0]
    normT = nuv[0, :, 0, :].T
    nuv9 = nuv[0].reshape(N, 9)

    jd, pts_s = pl.pallas_call(
        _prologue_body,
        out_shape=(
            jax.ShapeDtypeStruct((16, N), jnp.float32),
            jax.ShapeDtypeStruct((N, 3), jnp.float32),
        ),
    )(featT, ptsT, pts, normT,
      W_in1, b_in1.reshape(-1, 1), W_in2, b_in2.reshape(-1, 1),
      g_in.reshape(-1, 1), be_in.reshape(-1, 1))

    eye8 = jnp.eye(8, dtype=jnp.float32)
    w1k = jnp.kron(jnp.concatenate([Wc1, bc1[:, None]], axis=1), eye8)
    w2k = jnp.kron(jnp.concatenate([Wc2, bc2[:, None]], axis=1), eye8)

    pair = pl.pallas_call(
        _pair_body,
        grid=(N // BI,),
        in_specs=[
            pl.BlockSpec((64, 32), lambda i: (0, 0)),
            pl.BlockSpec((64, 72), lambda i: (0, 0)),
            pl.BlockSpec((BI, 3), lambda i: (i, 0)),
            pl.BlockSpec((BI, 9), lambda i: (i, 0)),
            pl.BlockSpec((16, N), lambda i: (0, 0)),
        ],
        out_specs=pl.BlockSpec((BI, 8), lambda i: (i, 0)),
        out_shape=jax.ShapeDtypeStruct((N, 8), jnp.float32),
        compiler_params=pltpu.CompilerParams(
            dimension_semantics=("arbitrary",)),
    )(w1k, w2k, pts_s, nuv9, jd)

    outT = pl.pallas_call(
        _epilogue_body,
        out_shape=jax.ShapeDtypeStruct((16, N), jnp.float32),
    )(pair.T, W_out1, b_out1.reshape(-1, 1), W_out2, b_out2.reshape(-1, 1),
      g_out.reshape(-1, 1), be_out.reshape(-1, 1))

    return outT.T[None]


# speedup vs baseline: 1.2480x; 1.2480x over previous
import math

import jax
import jax.numpy as jnp
from jax.experimental import pallas as pl
from jax.experimental.pallas import tpu as pltpu

N = 2048
BI = 8
BJ = 512
SCALE = 1.0 / math.sqrt(2.0)


def _lrelu(x):
    return jnp.where(x >= 0, x, 0.2 * x)


def _group_norm_rows(x, gamma, beta, groups, eps=1e-5):
    c = x.shape[0]
    per = c // groups
    outs = []
    for g in range(groups):
        sub = x[g * per:(g + 1) * per, :]
        m = jnp.mean(sub)
        v = jnp.mean((sub - m) ** 2)
        outs.append((sub - m) / jnp.sqrt(v + eps))
    y = jnp.concatenate(outs, axis=0)
    return y * gamma + beta


def _prologue_body(featT_ref, ptsT_ref, pts_ref, normT_ref,
                   w1_ref, b1_ref, w2_ref, b2_ref, g_ref, be_ref,
                   jd_ref, pts_out_ref):
    f = jnp.dot(w1_ref[...], featT_ref[...],
                preferred_element_type=jnp.float32) + b1_ref[...]
    f = _lrelu(f)
    f = jnp.dot(w2_ref[...], f, preferred_element_type=jnp.float32) + b2_ref[...]
    f = _lrelu(f)
    f = _group_norm_rows(f, g_ref[...], be_ref[...], groups=4)
    jd = jnp.concatenate([
        ptsT_ref[...] * SCALE,
        normT_ref[...],
        jnp.zeros((2, N), jnp.float32),
        f,
    ], axis=0)
    jd_ref[...] = jd
    pts_out_ref[...] = pts_ref[...] * SCALE


def _pair_body(wc1_ref, bc1_ref, wc2_ref, bc2_ref,
               pts_i_ref, nuv_i_ref, jd_ref, out_ref):
    xi = [pts_i_ref[:, c:c + 1] for c in range(3)]
    frame = [[nuv_i_ref[:, 3 * k + c:3 * k + c + 1] for c in range(3)]
             for k in range(3)]
    ni = frame[0]
    accs = [jnp.zeros((BI, BJ), jnp.float32) for _ in range(8)]
    for blk in range(N // BJ):
        sl = slice(blk * BJ, (blk + 1) * BJ)
        xj = [jd_ref[c:c + 1, sl] for c in range(3)]
        nj = [jd_ref[3 + c:4 + c, sl] for c in range(3)]
        dx = [xj[c] - xi[c] for c in range(3)]
        ndot = ni[0] * nj[0] + ni[1] * nj[1] + ni[2] * nj[2]
        sq = dx[0] * dx[0] + dx[1] * dx[1] + dx[2] * dx[2]
        t = 2.0 - ndot
        w = jnp.exp(-(sq * t * t))
        x_loc = [frame[k][0] * dx[0] + frame[k][1] * dx[1] + frame[k][2] * dx[2]
                 for k in range(3)]
        cuts = []
        for u in range(8):
            a = (x_loc[0] * wc1_ref[u, 0] + x_loc[1] * wc1_ref[u, 1]
                 + x_loc[2] * wc1_ref[u, 2] + bc1_ref[u])
            cuts.append(jnp.maximum(a, 0.0))
        for h in range(8):
            s = cuts[0] * wc2_ref[h, 0]
            for u in range(1, 8):
                s = s + cuts[u] * wc2_ref[h, u]
            hh = jnp.maximum(s + bc2_ref[h], 0.0)
            accs[h] = accs[h] + (w * hh) * jd_ref[8 + h:9 + h, sl]
    out_ref[...] = jnp.concatenate(
        [jnp.sum(a, axis=1, keepdims=True) for a in accs], axis=1)


def _epilogue_body(pT_ref, w1_ref, b1_ref, w2_ref, b2_ref, g_ref, be_ref,
                   out_ref):
    f = jnp.dot(w1_ref[...], pT_ref[...],
                preferred_element_type=jnp.float32) + b1_ref[...]
    f = _lrelu(f)
    f = jnp.dot(w2_ref[...], f, preferred_element_type=jnp.float32) + b2_ref[...]
    f = _lrelu(f)
    out_ref[...] = _group_norm_rows(f, g_ref[...], be_ref[...], groups=4)


def kernel(points, nuv, features, W_in1, b_in1, W_in2, b_in2, g_in, be_in,
           Wc1, bc1, Wc2, bc2, W_out1, b_out1, W_out2, b_out2, g_out, be_out):
    featT = features[0].T
    ptsT = points[0].T
    pts = points[0]
    normT = nuv[0, :, 0, :].T
    nuv9 = nuv[0].reshape(N, 9)

    jd, pts_s = pl.pallas_call(
        _prologue_body,
        out_shape=(
            jax.ShapeDtypeStruct((16, N), jnp.float32),
            jax.ShapeDtypeStruct((N, 3), jnp.float32),
        ),
    )(featT, ptsT, pts, normT,
      W_in1, b_in1.reshape(-1, 1), W_in2, b_in2.reshape(-1, 1),
      g_in.reshape(-1, 1), be_in.reshape(-1, 1))

    pair = pl.pallas_call(
        _pair_body,
        grid=(N // BI,),
        in_specs=[
            pl.BlockSpec(memory_space=pltpu.SMEM),
            pl.BlockSpec(memory_space=pltpu.SMEM),
            pl.BlockSpec(memory_space=pltpu.SMEM),
            pl.BlockSpec(memory_space=pltpu.SMEM),
            pl.BlockSpec((BI, 3), lambda i: (i, 0)),
            pl.BlockSpec((BI, 9), lambda i: (i, 0)),
            pl.BlockSpec((16, N), lambda i: (0, 0)),
        ],
        out_specs=pl.BlockSpec((BI, 8), lambda i: (i, 0)),
        out_shape=jax.ShapeDtypeStruct((N, 8), jnp.float32),
        compiler_params=pltpu.CompilerParams(
            dimension_semantics=("arbitrary",)),
    )(Wc1, bc1, Wc2, bc2, pts_s, nuv9, jd)

    outT = pl.pallas_call(
        _epilogue_body,
        out_shape=jax.ShapeDtypeStruct((16, N), jnp.float32),
    )(pair.T, W_out1, b_out1.reshape(-1, 1), W_out2, b_out2.reshape(-1, 1),
      g_out.reshape(-1, 1), be_out.reshape(-1, 1))

    return outT.T[None]
